# Initial kernel scaffold; baseline (speedup 1.0000x reference)
import jax
import jax.numpy as jnp
from jax.experimental import pallas as pl


def kernel(ent_embs, rels, neg_idx, rel_emb_weight):
    raise NotImplementedError("write your pallas kernel here")


# speedup vs baseline: 14.4230x; 14.4230x over previous
import jax
import jax.numpy as jnp
import numpy as np
from jax import lax
from jax.experimental import pallas as pl
from jax.experimental.pallas import tpu as pltpu
from jax.experimental.pallas import tpu_sc as plsc

_B = 4096
_D = 128
_NNEG = 64
_REG = 0.01

_info = plsc.get_sparse_core_info()
_NC = _info.num_cores
_NS = _info.num_subcores
_L = _info.num_lanes
_NW = _NC * _NS
_BPW = _B // _NW
_NV = _D // _L


def _sc_body(flat_hbm, rels_hbm, hidx_hbm, tidx_hbm, relw_hbm,
             neg_out, relrow_out,
             relrow_v, rels_v, hidx_v, tidx_v,
             hbuf0, hbuf1, tbuf0, tbuf1,
             scores_v,
             sem_h0, sem_h1, sem_t0, sem_t1, sem_rel):
    wid = lax.axis_index("s") * _NC + lax.axis_index("c")
    base = wid * _BPW

    pltpu.sync_copy(rels_hbm.at[pl.ds(base, _BPW)], rels_v)
    rel_cp = pltpu.async_copy(relw_hbm.at[rels_v], relrow_v, sem_rel)
    pltpu.sync_copy(hidx_hbm.at[pl.ds(base, _BPW)], hidx_v)
    pltpu.sync_copy(tidx_hbm.at[pl.ds(base, _BPW)], tidx_v)
    rel_cp.wait()

    hbufs = (hbuf0, hbuf1)
    tbufs = (tbuf0, tbuf1)
    sem_hs = (sem_h0, sem_h1)
    sem_ts = (sem_t0, sem_t1)

    def h_cp(bb, par):
        return pltpu.make_async_copy(flat_hbm.at[hidx_v.at[bb]],
                                     hbufs[par], sem_hs[par])

    def t_cp(bb, par):
        return pltpu.make_async_copy(flat_hbm.at[tidx_v.at[bb]],
                                     tbufs[par], sem_ts[par])

    h_cp(0, 0).start()
    t_cp(0, 0).start()
    h_cp(1, 1).start()
    t_cp(1, 1).start()

    zero = jnp.zeros((_L,), jnp.float32)
    lane = lax.iota(jnp.int32, _L)
    rots = [((lane + k) & (_L - 1)).reshape(_L, 1) for k in (8, 4, 2, 1)]
    _dnums = lax.GatherDimensionNumbers(
        offset_dims=(), collapsed_slice_dims=(0,), start_index_map=(0,))

    def lane_sum(x):
        for perm in rots:
            x = x + lax.gather(x, perm, _dnums, (1,),
                               mode=lax.GatherScatterMode.PROMISE_IN_BOUNDS)
        return x

    @pl.loop(0, _BPW, step=2)
    def _b_loop(b):
        for par in range(2):
            bb = b + par
            h_cp(bb, par).wait()
            t_cp(bb, par).wait()
            hbuf = hbufs[par]
            tbuf = tbufs[par]

            relv = [relrow_v[bb, pl.ds(v * _L, _L)] for v in range(_NV)]

            for c in range(_NNEG // _L):
                @pl.loop(0, _L, init_carry=zero)
                def pending(n, pending):
                    nn = c * _L + n
                    nacc = zero
                    for v in range(_NV):
                        hv = hbuf[nn, pl.ds(v * _L, _L)]
                        tv = tbuf[nn, pl.ds(v * _L, _L)]
                        nacc = nacc + hv * relv[v] * tv
                    return jnp.where(lane == n, lane_sum(nacc), pending)

                scores_v[bb, pl.ds(c * _L, _L)] = pending

            @pl.when(bb + 2 < _BPW)
            def _prefetch():
                h_cp(bb + 2, par).start()
                t_cp(bb + 2, par).start()

    pltpu.sync_copy(scores_v, neg_out.at[pl.ds(base, _BPW)])
    pltpu.sync_copy(relrow_v, relrow_out.at[pl.ds(base, _BPW)])


def _sc_scores(flat, rels, hidx, tidx, relw):
    mesh = plsc.VectorSubcoreMesh(core_axis_name="c", subcore_axis_name="s")
    return pl.kernel(
        _sc_body,
        out_type=(
            jax.ShapeDtypeStruct((_B, _NNEG), jnp.float32),
            jax.ShapeDtypeStruct((_B, _D), jnp.float32),
        ),
        mesh=mesh,
        scratch_types=[
            pltpu.VMEM((_BPW, _D), jnp.float32),
            pltpu.VMEM((_BPW,), jnp.int32),
            pltpu.VMEM((_BPW, _NNEG), jnp.int32),
            pltpu.VMEM((_BPW, _NNEG), jnp.int32),
            pltpu.VMEM((_NNEG, _D), jnp.float32),
            pltpu.VMEM((_NNEG, _D), jnp.float32),
            pltpu.VMEM((_NNEG, _D), jnp.float32),
            pltpu.VMEM((_NNEG, _D), jnp.float32),
            pltpu.VMEM((_BPW, _NNEG), jnp.float32),
            pltpu.SemaphoreType.DMA,
            pltpu.SemaphoreType.DMA,
            pltpu.SemaphoreType.DMA,
            pltpu.SemaphoreType.DMA,
            pltpu.SemaphoreType.DMA,
        ],
    )(flat, rels, hidx, tidx, relw)


def _log_sigmoid(x):
    return jnp.minimum(x, 0.0) - jnp.log1p(jnp.exp(-jnp.abs(x)))


def _tc_body(neg_ref, relrow_ref, ent_ref, out_ref):
    neg = neg_ref[...]
    rel = relrow_ref[...]
    ent = ent_ref[...]
    heads = ent[:, 0, :]
    tails = ent[:, 1, :]
    pos = jnp.sum(heads * rel * tails, axis=-1)
    neg_loss = -jnp.sum(_log_sigmoid(-neg)) / (_B * _NNEG)
    pos_loss = -jnp.sum(_log_sigmoid(pos)) / _B
    model_loss = (pos_loss + neg_loss) * 0.5
    ent_sq = jnp.sum(ent * ent) / (_B * _D)
    rel_sq = jnp.sum(rel * rel) / (_B * _D)
    reg = _REG * ((ent_sq + rel_sq) / 3.0)
    out_ref[...] = jnp.full((1, 1), 0.0, jnp.float32) + model_loss + reg


def _tc_finish(neg_scores, relrows, ent_embs):
    out = pl.pallas_call(
        _tc_body,
        out_shape=jax.ShapeDtypeStruct((1, 1), jnp.float32),
    )(neg_scores, relrows, ent_embs)
    return out[0, 0]


def kernel(ent_embs, rels, neg_idx, rel_emb_weight):
    ent = ent_embs.astype(jnp.float32)
    flat = ent.reshape(2 * _B, _D)
    rels1 = rels.reshape(_B).astype(jnp.int32)
    hidx = neg_idx[:, :, 0].astype(jnp.int32)
    tidx = neg_idx[:, :, 1].astype(jnp.int32)
    relw = rel_emb_weight.astype(jnp.float32)
    neg_scores, relrows = _sc_scores(flat, rels1, hidx, tidx, relw)
    return _tc_finish(neg_scores, relrows, ent)
